# baseline (speedup 1.0000x reference)
import functools

import jax
import jax.numpy as jnp
from jax import lax
from jax.experimental import pallas as pl
from jax.experimental.pallas import tpu as pltpu
from jax.experimental.pallas import tpu_sc as plsc

N_NODES = 10000
N_EDGES = 160000
V = 32768
B = 1024
D_IN = 300
NOISE = 300
NGH = 512
FEA = 512

D1P = 112
D2P = 64
NP = 10240

NW = 32
SUB = 80
NSUB = 64
EW = SUB * NSUB
EPAD = NW * EW

JW = 640
JG = JW // 16
BW = B // NW

_mesh = plsc.VectorSubcoreMesh(core_axis_name="c", subcore_axis_name="s")
_sc_params = pltpu.CompilerParams(use_tc_tiling_on_sc=False)


def _mm1_body(x_ref, w_ref, o_ref):
    o_ref[...] = lax.dot_general(
        x_ref[...], w_ref[...], (((1,), (0,)), ((), ())),
        preferred_element_type=jnp.float32)


def _pad_body(n_ref, o_ref):
    o_ref[...] = jnp.pad(n_ref[...], ((0, 0), (0, D2P - n_ref.shape[1])))


def _mm2_body(p0_ref, p1_ref, w_ref, o_ref):
    h = jnp.maximum(p0_ref[0] + p1_ref[0], 0.0)
    o_ref[...] = lax.dot_general(
        h, w_ref[...], (((1,), (0,)), ((), ())),
        preferred_element_type=jnp.float32)


def _route_idx_body(ii_ref, upd_ref, cor_ref, o_ref):
    i32 = jnp.int32
    ii = ii_ref[...]
    lane = lax.broadcasted_iota(i32, (1, 128), 1)
    neg = jnp.full((B, 128), -1, i32)

    def pass1(g, acc):
        jval = g * 128 + lane
        upd_row = upd_ref[pl.ds(g, 1), :]
        cor_row = cor_ref[pl.ds(g, 1), :]
        hit = jnp.logical_and(ii == upd_row, jval < N_NODES)
        pack = jval * 16384 + cor_row
        return jnp.maximum(acc, jnp.where(hit, pack, neg))

    jacc = lax.fori_loop(0, NP // 128, pass1, neg)
    m = jnp.max(jacc, axis=1, keepdims=True)
    cj = jnp.where(m >= 0, jnp.bitwise_and(m, 16383), 0)
    o_ref[...] = jnp.where(lane == 0, jnp.broadcast_to(m, (B, 128)),
                           jnp.where(lane == 1, jnp.broadcast_to(cj, (B, 128)),
                                     0))


def _mlp_body(noise_ref, ri_ref, gn2_ref, rt_ref, a_ref, bT_ref, b1_ref,
              w2T_ref, b2_ref, h_ref, ce_ref):
    dn = (((1,), (0,)), ((), ()))
    sel = ri_ref[:, 0:1] >= 0
    ce = jnp.where(sel, rt_ref[0] + rt_ref[1], gn2_ref[...])
    ce_ref[...] = ce
    t = (lax.dot_general(noise_ref[...], a_ref[...], dn,
                         preferred_element_type=jnp.float32)
         + lax.dot_general(ce, bT_ref[...], dn,
                           preferred_element_type=jnp.float32)
         + b1_ref[...])
    t = jnp.where(t >= 0.0, t, 0.2 * t)
    t2 = lax.dot_general(t, w2T_ref[...], dn,
                         preferred_element_type=jnp.float32) + b2_ref[...]
    h_ref[...] = jnp.maximum(t2, 0.0)


def _make_spmm(D, routed):
    ZR = 16
    RT = NP // 16
    FL = 80
    NB = 2
    NT = NSUB // NB
    RB = B // 16

    if routed:
        out_type = (jax.ShapeDtypeStruct((2, B, D), jnp.float32),
                    jax.ShapeDtypeStruct((B, D), jnp.float32))
        extra_scratch = [
            pltpu.VMEM((BW,), jnp.int32),
            pltpu.VMEM((RB,), jnp.int32),
            pltpu.VMEM((BW, D), jnp.float32),
            pltpu.VMEM((RB, D), jnp.float32),
            pltpu.SemaphoreType.DMA,
        ]
    else:
        out_type = jax.ShapeDtypeStruct((2, NP, D), jnp.float32)
        extra_scratch = []

    @functools.partial(
        pl.kernel,
        out_type=out_type,
        mesh=_mesh,
        compiler_params=_sc_params,
        scratch_types=[
            pltpu.VMEM((NSUB, SUB), jnp.int32),
            pltpu.VMEM((NSUB, SUB), jnp.int32),
            pltpu.VMEM((EW,), jnp.float32),
            pltpu.VMEM((NB, SUB, D), jnp.float32),
            pltpu.VMEM((NB, SUB, D), jnp.float32),
            pltpu.VMEM((ZR, D), jnp.float32),
            pltpu.VMEM_SHARED((NP, D), jnp.float32),
            pltpu.SemaphoreType.DMA,
            pltpu.SemaphoreType.DMA,
            pltpu.SemaphoreType.DMA,
            pltpu.SemaphoreType.DMA,
        ] + extra_scratch,
    )
    def spmm(table, cols2d, rows2d, vals1d, *rest):
        if routed:
            (ii2d, cj2d, n2vp, rout, nout,
             colv, rowv, valv, gbuf, sbuf, zbuf, acc, gs0, gs1, ss0, ss1,
             iiv, cjv, gn2, gac, rsem) = rest
        else:
            (out,
             colv, rowv, valv, gbuf, sbuf, zbuf, acc, gs0, gs1, ss0, ss1
             ) = rest
        cid = lax.axis_index("c")
        sid = lax.axis_index("s")
        wid = cid * 16 + sid
        gsem = (gs0, gs1)
        ssem = (ss0, ss1)

        pltpu.sync_copy(cols2d.at[pl.ds(wid * NSUB, NSUB)], colv)
        pltpu.sync_copy(rows2d.at[pl.ds(wid * NSUB, NSUB)], rowv)
        pltpu.sync_copy(vals1d.at[pl.ds(wid * EW, EW)], valv)

        if routed:
            pltpu.sync_copy(ii2d.at[wid], iiv)
            pltpu.async_copy(n2vp.at[iiv], gn2, rsem)

        zero16 = jnp.zeros((16,), jnp.float32)
        for r in range(ZR):
            for d in range(D // 16):
                zbuf[r, pl.ds(16 * d, 16)] = zero16
        for i in range(RT // ZR):
            pltpu.sync_copy(zbuf, acc.at[pl.ds(sid * RT + i * ZR, ZR)])
        plsc.subcore_barrier()

        dn = lax.GatherDimensionNumbers(
            offset_dims=(), collapsed_slice_dims=(0,), start_index_map=(0,))

        def start_gather(i, b):
            pltpu.async_copy(table.at[colv.at[i]], gbuf.at[b], gsem[b])

        def wait_gather(i, b):
            pltpu.make_async_copy(table.at[colv.at[i]], gbuf.at[b],
                                  gsem[b]).wait()

        def start_scatter(i, b):
            pltpu.async_copy(sbuf.at[b], acc.at[rowv.at[i]], ssem[b],
                             add=True)

        def wait_scatter(i, b):
            pltpu.make_async_copy(sbuf.at[b], acc.at[rowv.at[i]],
                                  ssem[b]).wait()

        def scale(i, b):
            for k in range(SUB // 16):
                v16 = valv[pl.ds(i * SUB + k * 16, 16)]
                for l in range(16):
                    bc = lax.gather(
                        v16, jnp.full((16, 1), l, jnp.int32), dn, (1,),
                        mode=lax.GatherScatterMode.PROMISE_IN_BOUNDS)
                    e = k * 16 + l
                    for d in range(D // 16):
                        sl = pl.ds(16 * d, 16)
                        sbuf[b, e, sl] = gbuf[b, e, sl] * bc

        for b in range(NB):
            start_gather(b, b)
        for b in range(NB):
            wait_gather(b, b)
            scale(b, b)
            start_gather(b + NB, b)
            start_scatter(b, b)

        def main(t, c):
            for b in range(NB):
                i = t * NB + b
                wait_gather(i, b)
                wait_scatter(i, b)
                scale(i, b)
                start_gather(i + NB, b)
                start_scatter(i, b)
            return c

        lax.fori_loop(1, NT - 1, main, 0)

        for b in range(NB):
            i = (NT - 1) * NB + b
            wait_gather(i, b)
            wait_scatter(i, b)
            scale(i, b)
            start_scatter(i, b)
        for b in range(NB):
            wait_scatter(0, b)
        plsc.subcore_barrier()

        if routed:
            pltpu.make_async_copy(n2vp.at[iiv], gn2, rsem).wait()
            pltpu.sync_copy(gn2, nout.at[pl.ds(wid * BW, BW)])
            pltpu.sync_copy(cj2d.at[sid], cjv)
            pltpu.async_copy(acc.at[cjv], gac, rsem).wait()
            pltpu.sync_copy(gac, rout.at[cid, pl.ds(sid * RB, RB)])
        else:
            for i in range(RT // FL):
                base = sid * RT + i * FL
                pltpu.sync_copy(acc.at[pl.ds(base, FL)], sbuf.at[0])
                pltpu.sync_copy(sbuf.at[0], out.at[cid, pl.ds(base, FL)])

    return spmm


_spmm_d1 = _make_spmm(D1P, routed=False)
_spmm_d2 = _make_spmm(D2P, routed=True)


def kernel(noise, input_cls_feat, adj_values, n2v, input_index, adj_indices,
           update_idx, corresp_idx, W1, W2, fc1_w, fc1_b, fc2_w, fc2_b):
    f32 = jnp.float32
    i32 = jnp.int32

    rows = adj_indices[0].astype(i32)
    cols = adj_indices[1].astype(i32)
    vals = adj_values.astype(f32)

    pad = EPAD - N_EDGES
    spread = (jnp.arange(pad, dtype=i32) * 61) % N_NODES
    cols2d = jnp.concatenate([cols, spread]).reshape(EPAD // SUB, SUB)
    rows2d = jnp.concatenate([rows, spread]).reshape(EPAD // SUB, SUB)
    vals1d = jnp.concatenate([vals, jnp.zeros((pad,), f32)])

    W1p = jnp.pad(W1.astype(f32), ((0, 0), (0, D1P - W1.shape[1])))
    W2p = jnp.pad(W2.astype(f32), ((0, D1P - W2.shape[0]),
                                   (0, D2P - W2.shape[1])))

    support1 = pl.pallas_call(
        _mm1_body,
        grid=(10,),
        in_specs=[pl.BlockSpec((1000, D_IN), lambda i: (i, 0)),
                  pl.BlockSpec((D_IN, D1P), lambda i: (0, 0))],
        out_specs=pl.BlockSpec((1000, D1P), lambda i: (i, 0)),
        out_shape=jax.ShapeDtypeStruct((N_NODES, D1P), f32),
    )(input_cls_feat.astype(f32), W1p)
    n2vp = pl.pallas_call(
        _pad_body,
        grid=(8,),
        in_specs=[pl.BlockSpec((V // 8, n2v.shape[1]), lambda i: (i, 0))],
        out_specs=pl.BlockSpec((V // 8, D2P), lambda i: (i, 0)),
        out_shape=jax.ShapeDtypeStruct((V, D2P), f32),
    )(n2v.astype(f32))

    part1 = _spmm_d1(support1, cols2d, rows2d, vals1d)

    support2 = pl.pallas_call(
        _mm2_body,
        grid=(10,),
        in_specs=[pl.BlockSpec((1, 1024, D1P), lambda i: (0, i, 0)),
                  pl.BlockSpec((1, 1024, D1P), lambda i: (1, i, 0)),
                  pl.BlockSpec((D1P, D2P), lambda i: (0, 0))],
        out_specs=pl.BlockSpec((1024, D2P), lambda i: (i, 0)),
        out_shape=jax.ShapeDtypeStruct((NP, D2P), f32),
    )(part1, part1, W2p)

    ii_b = jnp.broadcast_to(input_index.astype(i32)[:, None], (B, 128))
    upd2d = jnp.pad(update_idx.astype(i32), (0, NP - N_NODES)
                    ).reshape(NP // 128, 128)
    cor2d = jnp.pad(corresp_idx.astype(i32), (0, NP - N_NODES)
                    ).reshape(NP // 128, 128)
    route_idx = pl.pallas_call(
        _route_idx_body,
        in_specs=[pl.BlockSpec((B, 128), lambda: (0, 0)),
                  pl.BlockSpec((NP // 128, 128), lambda: (0, 0)),
                  pl.BlockSpec((NP // 128, 128), lambda: (0, 0))],
        out_specs=pl.BlockSpec((B, 128), lambda: (0, 0)),
        out_shape=jax.ShapeDtypeStruct((B, 128), i32),
    )(ii_b, upd2d, cor2d)
    cj2d = route_idx[:, 1].reshape(16, B // 16)

    ii2d = input_index.astype(i32).reshape(NW, BW)
    gcls_planes, gn2_rows = _spmm_d2(support2, cols2d, rows2d, vals1d,
                                     ii2d, cj2d, n2vp)

    fc1aT = fc1_w[:, :NOISE].T.astype(f32)
    fc1bT = jnp.pad(fc1_w[:, NOISE:].T.astype(f32),
                    ((0, D2P - (fc1_w.shape[1] - NOISE)), (0, 0)))
    h, cls_embed_p = pl.pallas_call(
        _mlp_body,
        in_specs=[pl.BlockSpec((B, NOISE), lambda: (0, 0)),
                  pl.BlockSpec((B, 128), lambda: (0, 0)),
                  pl.BlockSpec((B, D2P), lambda: (0, 0)),
                  pl.BlockSpec((2, B, D2P), lambda: (0, 0, 0)),
                  pl.BlockSpec((NOISE, NGH), lambda: (0, 0)),
                  pl.BlockSpec((D2P, NGH), lambda: (0, 0)),
                  pl.BlockSpec((1, NGH), lambda: (0, 0)),
                  pl.BlockSpec((NGH, FEA), lambda: (0, 0)),
                  pl.BlockSpec((1, FEA), lambda: (0, 0))],
        out_specs=(pl.BlockSpec((B, FEA), lambda: (0, 0)),
                   pl.BlockSpec((B, D2P), lambda: (0, 0))),
        out_shape=(jax.ShapeDtypeStruct((B, FEA), f32),
                   jax.ShapeDtypeStruct((B, D2P), f32)),
    )(noise.astype(f32), route_idx, gn2_rows, gcls_planes, fc1aT, fc1bT,
      fc1_b.reshape(1, NGH).astype(f32), fc2_w.T.astype(f32),
      fc2_b.reshape(1, FEA).astype(f32))

    cls_embed = cls_embed_p[:, :n2v.shape[1]]
    return (h, cls_embed)


# speedup vs baseline: 1.0304x; 1.0304x over previous
import functools

import jax
import jax.numpy as jnp
from jax import lax
from jax.experimental import pallas as pl
from jax.experimental.pallas import tpu as pltpu
from jax.experimental.pallas import tpu_sc as plsc

N_NODES = 10000
N_EDGES = 160000
V = 32768
B = 1024
D_IN = 300
NOISE = 300
NGH = 512
FEA = 512

D1P = 112
D2P = 64
NP = 10240

NW = 32
SUB = 80
NSUB = 64
EW = SUB * NSUB
EPAD = NW * EW

JW = 640
JG = JW // 16
BW = B // NW

_mesh = plsc.VectorSubcoreMesh(core_axis_name="c", subcore_axis_name="s")
_sc_params = pltpu.CompilerParams(use_tc_tiling_on_sc=False)


def _mm1_body(x_ref, w_ref, o_ref):
    o_ref[...] = lax.dot_general(
        x_ref[...], w_ref[...], (((1,), (0,)), ((), ())),
        preferred_element_type=jnp.float32).astype(jnp.bfloat16)


def _pad_body(n_ref, o_ref):
    o_ref[...] = jnp.pad(n_ref[...], ((0, 0), (0, D2P - n_ref.shape[1])))


def _mm2_body(p0_ref, p1_ref, w_ref, o_ref):
    h = jnp.maximum(p0_ref[0] + p1_ref[0], 0.0)
    o_ref[...] = lax.dot_general(
        h, w_ref[...], (((1,), (0,)), ((), ())),
        preferred_element_type=jnp.float32).astype(jnp.bfloat16)


def _route_idx_body(ii_ref, upd_ref, cor_ref, o_ref):
    i32 = jnp.int32
    ii = ii_ref[...]
    lane = lax.broadcasted_iota(i32, (1, 128), 1)
    neg = jnp.full((B, 128), -1, i32)

    def pass1(g, acc):
        jval = g * 128 + lane
        upd_row = upd_ref[pl.ds(g, 1), :]
        cor_row = cor_ref[pl.ds(g, 1), :]
        hit = jnp.logical_and(ii == upd_row, jval < N_NODES)
        pack = jval * 16384 + cor_row
        return jnp.maximum(acc, jnp.where(hit, pack, neg))

    jacc = lax.fori_loop(0, NP // 128, pass1, neg)
    m = jnp.max(jacc, axis=1, keepdims=True)
    cj = jnp.where(m >= 0, jnp.bitwise_and(m, 16383), 0)
    o_ref[...] = jnp.where(lane == 0, jnp.broadcast_to(m, (B, 128)),
                           jnp.where(lane == 1, jnp.broadcast_to(cj, (B, 128)),
                                     0))


def _mlp_body(noise_ref, ri_ref, gn2_ref, rt_ref, a_ref, bT_ref, b1_ref,
              w2T_ref, b2_ref, h_ref, ce_ref):
    dn = (((1,), (0,)), ((), ()))
    sel = ri_ref[:, 0:1] >= 0
    ce = jnp.where(sel, rt_ref[0] + rt_ref[1], gn2_ref[...])
    ce_ref[...] = ce
    t = (lax.dot_general(noise_ref[...], a_ref[...], dn,
                         preferred_element_type=jnp.float32)
         + lax.dot_general(ce, bT_ref[...], dn,
                           preferred_element_type=jnp.float32)
         + b1_ref[...])
    t = jnp.where(t >= 0.0, t, 0.2 * t)
    t2 = lax.dot_general(t, w2T_ref[...], dn,
                         preferred_element_type=jnp.float32) + b2_ref[...]
    h_ref[...] = jnp.maximum(t2, 0.0)


def _make_spmm(D, routed):
    ZR = 16
    RT = NP // 16
    FL = 80
    NB = 2
    NT = NSUB // NB
    RB = B // 16

    if routed:
        out_type = (jax.ShapeDtypeStruct((2, B, D), jnp.float32),
                    jax.ShapeDtypeStruct((B, D), jnp.float32))
        extra_scratch = [
            pltpu.VMEM((BW,), jnp.int32),
            pltpu.VMEM((RB,), jnp.int32),
            pltpu.VMEM((BW, D), jnp.float32),
            pltpu.VMEM((RB, D), jnp.float32),
            pltpu.SemaphoreType.DMA,
        ]
    else:
        out_type = jax.ShapeDtypeStruct((2, NP, D), jnp.float32)
        extra_scratch = []

    @functools.partial(
        pl.kernel,
        out_type=out_type,
        mesh=_mesh,
        compiler_params=_sc_params,
        scratch_types=[
            pltpu.VMEM((NSUB, SUB), jnp.int32),
            pltpu.VMEM((NSUB, SUB), jnp.int32),
            pltpu.VMEM((EW,), jnp.float32),
            pltpu.VMEM((NB, SUB, D), jnp.bfloat16),
            pltpu.VMEM((NB, SUB, D), jnp.float32),
            pltpu.VMEM((ZR, D), jnp.float32),
            pltpu.VMEM_SHARED((NP, D), jnp.float32),
            pltpu.SemaphoreType.DMA,
            pltpu.SemaphoreType.DMA,
            pltpu.SemaphoreType.DMA,
            pltpu.SemaphoreType.DMA,
        ] + extra_scratch,
    )
    def spmm(table, cols2d, rows2d, vals1d, *rest):
        if routed:
            (ii2d, cj2d, n2vp, rout, nout,
             colv, rowv, valv, gbuf, sbuf, zbuf, acc, gs0, gs1, ss0, ss1,
             iiv, cjv, gn2, gac, rsem) = rest
        else:
            (out,
             colv, rowv, valv, gbuf, sbuf, zbuf, acc, gs0, gs1, ss0, ss1
             ) = rest
        cid = lax.axis_index("c")
        sid = lax.axis_index("s")
        wid = cid * 16 + sid
        gsem = (gs0, gs1)
        ssem = (ss0, ss1)

        pltpu.sync_copy(cols2d.at[pl.ds(wid * NSUB, NSUB)], colv)
        pltpu.sync_copy(rows2d.at[pl.ds(wid * NSUB, NSUB)], rowv)
        pltpu.sync_copy(vals1d.at[pl.ds(wid * EW, EW)], valv)

        if routed:
            pltpu.sync_copy(ii2d.at[wid], iiv)
            pltpu.async_copy(n2vp.at[iiv], gn2, rsem)

        zero16 = jnp.zeros((16,), jnp.float32)
        for r in range(ZR):
            for d in range(D // 16):
                zbuf[r, pl.ds(16 * d, 16)] = zero16
        for i in range(RT // ZR):
            pltpu.sync_copy(zbuf, acc.at[pl.ds(sid * RT + i * ZR, ZR)])
        plsc.subcore_barrier()

        dn = lax.GatherDimensionNumbers(
            offset_dims=(), collapsed_slice_dims=(0,), start_index_map=(0,))

        def start_gather(i, b):
            pltpu.async_copy(table.at[colv.at[i]], gbuf.at[b], gsem[b])

        def wait_gather(i, b):
            pltpu.make_async_copy(table.at[colv.at[i]], gbuf.at[b],
                                  gsem[b]).wait()

        def start_scatter(i, b):
            pltpu.async_copy(sbuf.at[b], acc.at[rowv.at[i]], ssem[b],
                             add=True)

        def wait_scatter(i, b):
            pltpu.make_async_copy(sbuf.at[b], acc.at[rowv.at[i]],
                                  ssem[b]).wait()

        def scale(i, b):
            for k in range(SUB // 16):
                v16 = valv[pl.ds(i * SUB + k * 16, 16)]
                for l in range(16):
                    bc = lax.gather(
                        v16, jnp.full((16, 1), l, jnp.int32), dn, (1,),
                        mode=lax.GatherScatterMode.PROMISE_IN_BOUNDS)
                    e = k * 16 + l
                    for d in range(D // 16):
                        sl = pl.ds(16 * d, 16)
                        sbuf[b, e, sl] = gbuf[b, e, sl].astype(
                            jnp.float32) * bc

        for b in range(NB):
            start_gather(b, b)
        for b in range(NB):
            wait_gather(b, b)
            scale(b, b)
            start_gather(b + NB, b)
            start_scatter(b, b)

        def main(t, c):
            for b in range(NB):
                i = t * NB + b
                wait_gather(i, b)
                wait_scatter(i, b)
                scale(i, b)
                start_gather(i + NB, b)
                start_scatter(i, b)
            return c

        lax.fori_loop(1, NT - 1, main, 0)

        for b in range(NB):
            i = (NT - 1) * NB + b
            wait_gather(i, b)
            wait_scatter(i, b)
            scale(i, b)
            start_scatter(i, b)
        for b in range(NB):
            wait_scatter(0, b)
        plsc.subcore_barrier()

        if routed:
            pltpu.make_async_copy(n2vp.at[iiv], gn2, rsem).wait()
            pltpu.sync_copy(gn2, nout.at[pl.ds(wid * BW, BW)])
            pltpu.sync_copy(cj2d.at[sid], cjv)
            pltpu.async_copy(acc.at[cjv], gac, rsem).wait()
            pltpu.sync_copy(gac, rout.at[cid, pl.ds(sid * RB, RB)])
        else:
            for i in range(RT // FL):
                base = sid * RT + i * FL
                pltpu.sync_copy(acc.at[pl.ds(base, FL)], sbuf.at[0])
                pltpu.sync_copy(sbuf.at[0], out.at[cid, pl.ds(base, FL)])

    return spmm


_spmm_d1 = _make_spmm(D1P, routed=False)
_spmm_d2 = _make_spmm(D2P, routed=True)


def kernel(noise, input_cls_feat, adj_values, n2v, input_index, adj_indices,
           update_idx, corresp_idx, W1, W2, fc1_w, fc1_b, fc2_w, fc2_b):
    f32 = jnp.float32
    i32 = jnp.int32

    rows = adj_indices[0].astype(i32)
    cols = adj_indices[1].astype(i32)
    vals = adj_values.astype(f32)

    pad = EPAD - N_EDGES
    spread = (jnp.arange(pad, dtype=i32) * 61) % N_NODES
    cols2d = jnp.concatenate([cols, spread]).reshape(EPAD // SUB, SUB)
    rows2d = jnp.concatenate([rows, spread]).reshape(EPAD // SUB, SUB)
    vals1d = jnp.concatenate([vals, jnp.zeros((pad,), f32)])

    W1p = jnp.pad(W1.astype(f32), ((0, 0), (0, D1P - W1.shape[1])))
    W2p = jnp.pad(W2.astype(f32), ((0, D1P - W2.shape[0]),
                                   (0, D2P - W2.shape[1])))

    support1 = pl.pallas_call(
        _mm1_body,
        grid=(10,),
        in_specs=[pl.BlockSpec((1000, D_IN), lambda i: (i, 0)),
                  pl.BlockSpec((D_IN, D1P), lambda i: (0, 0))],
        out_specs=pl.BlockSpec((1000, D1P), lambda i: (i, 0)),
        out_shape=jax.ShapeDtypeStruct((N_NODES, D1P), jnp.bfloat16),
    )(input_cls_feat.astype(f32), W1p)
    n2vp = pl.pallas_call(
        _pad_body,
        grid=(8,),
        in_specs=[pl.BlockSpec((V // 8, n2v.shape[1]), lambda i: (i, 0))],
        out_specs=pl.BlockSpec((V // 8, D2P), lambda i: (i, 0)),
        out_shape=jax.ShapeDtypeStruct((V, D2P), f32),
    )(n2v.astype(f32))

    part1 = _spmm_d1(support1, cols2d, rows2d, vals1d)

    support2 = pl.pallas_call(
        _mm2_body,
        grid=(10,),
        in_specs=[pl.BlockSpec((1, 1024, D1P), lambda i: (0, i, 0)),
                  pl.BlockSpec((1, 1024, D1P), lambda i: (1, i, 0)),
                  pl.BlockSpec((D1P, D2P), lambda i: (0, 0))],
        out_specs=pl.BlockSpec((1024, D2P), lambda i: (i, 0)),
        out_shape=jax.ShapeDtypeStruct((NP, D2P), jnp.bfloat16),
    )(part1, part1, W2p)

    ii_b = jnp.broadcast_to(input_index.astype(i32)[:, None], (B, 128))
    upd2d = jnp.pad(update_idx.astype(i32), (0, NP - N_NODES)
                    ).reshape(NP // 128, 128)
    cor2d = jnp.pad(corresp_idx.astype(i32), (0, NP - N_NODES)
                    ).reshape(NP // 128, 128)
    route_idx = pl.pallas_call(
        _route_idx_body,
        in_specs=[pl.BlockSpec((B, 128), lambda: (0, 0)),
                  pl.BlockSpec((NP // 128, 128), lambda: (0, 0)),
                  pl.BlockSpec((NP // 128, 128), lambda: (0, 0))],
        out_specs=pl.BlockSpec((B, 128), lambda: (0, 0)),
        out_shape=jax.ShapeDtypeStruct((B, 128), i32),
    )(ii_b, upd2d, cor2d)
    cj2d = route_idx[:, 1].reshape(16, B // 16)

    ii2d = input_index.astype(i32).reshape(NW, BW)
    gcls_planes, gn2_rows = _spmm_d2(support2, cols2d, rows2d, vals1d,
                                     ii2d, cj2d, n2vp)

    fc1aT = fc1_w[:, :NOISE].T.astype(f32)
    fc1bT = jnp.pad(fc1_w[:, NOISE:].T.astype(f32),
                    ((0, D2P - (fc1_w.shape[1] - NOISE)), (0, 0)))
    h, cls_embed_p = pl.pallas_call(
        _mlp_body,
        in_specs=[pl.BlockSpec((B, NOISE), lambda: (0, 0)),
                  pl.BlockSpec((B, 128), lambda: (0, 0)),
                  pl.BlockSpec((B, D2P), lambda: (0, 0)),
                  pl.BlockSpec((2, B, D2P), lambda: (0, 0, 0)),
                  pl.BlockSpec((NOISE, NGH), lambda: (0, 0)),
                  pl.BlockSpec((D2P, NGH), lambda: (0, 0)),
                  pl.BlockSpec((1, NGH), lambda: (0, 0)),
                  pl.BlockSpec((NGH, FEA), lambda: (0, 0)),
                  pl.BlockSpec((1, FEA), lambda: (0, 0))],
        out_specs=(pl.BlockSpec((B, FEA), lambda: (0, 0)),
                   pl.BlockSpec((B, D2P), lambda: (0, 0))),
        out_shape=(jax.ShapeDtypeStruct((B, FEA), f32),
                   jax.ShapeDtypeStruct((B, D2P), f32)),
    )(noise.astype(f32), route_idx, gn2_rows, gcls_planes, fc1aT, fc1bT,
      fc1_b.reshape(1, NGH).astype(f32), fc2_w.T.astype(f32),
      fc2_b.reshape(1, FEA).astype(f32))

    cls_embed = cls_embed_p[:, :n2v.shape[1]]
    return (h, cls_embed)
